# baseline (speedup 1.0000x reference)
import functools

import jax
import jax.numpy as jnp
from jax import lax
from jax.experimental import pallas as pl
from jax.experimental.pallas import tpu as pltpu
from jax.experimental.pallas import tpu_sc as plsc


def _scores_argmin_body(x_ref, cb_ref, x2_ref, c2_ref, idx_ref):
    xb = x_ref[0]
    s = lax.dot_general(cb_ref[...], xb.astype(jnp.bfloat16),
                        dimension_numbers=(((1,), (0,)), ((), ())),
                        preferred_element_type=jnp.float32)
    dist = jnp.sqrt(jnp.maximum((x2_ref[0] - 2.0 * s) + c2_ref[...], 0.0))
    m = jnp.min(dist, axis=0, keepdims=True)
    kio = lax.broadcasted_iota(jnp.int32, dist.shape, 0)
    big = jnp.int32(dist.shape[0])
    idx = jnp.min(jnp.where(dist == m, kio, big), axis=0)
    idx_ref[0, 0, :] = idx


def _scores_argmin(xr, cb16, x2, c2, bn):
    b, c, hw = xr.shape
    k = cb16.shape[0]
    nblk = (b * hw) // bn
    per_b = hw // bn
    return pl.pallas_call(
        _scores_argmin_body,
        grid=(nblk,),
        in_specs=[
            pl.BlockSpec((1, c, bn), lambda i: (i // per_b, 0, i % per_b)),
            pl.BlockSpec((k, c), lambda i: (0, 0)),
            pl.BlockSpec((1, 1, bn), lambda i: (i, 0, 0)),
            pl.BlockSpec((k, 1), lambda i: (0, 0)),
        ],
        out_specs=pl.BlockSpec((1, 1, bn), lambda i: (i, 0, 0)),
        out_shape=jax.ShapeDtypeStruct((nblk, 1, bn), jnp.int32),
    )(xr, cb16, x2, c2)


def _make_codebook_gather(k, c, n):
    info = plsc.get_sparse_core_info()
    nw = info.num_cores * info.num_subcores
    bpw = n // nw
    chunk = 128
    nch = bpw // chunk
    mesh = plsc.VectorSubcoreMesh(core_axis_name="c", subcore_axis_name="s")

    @functools.partial(
        pl.kernel,
        mesh=mesh,
        out_type=jax.ShapeDtypeStruct((n, c), jnp.float32),
        scratch_types=[
            pltpu.VMEM((chunk,), jnp.int32),
            pltpu.VMEM((chunk, c), jnp.float32),
            pltpu.SemaphoreType.DMA,
        ],
    )
    def gather(table_hbm, idx_hbm, out_hbm, idx_v, rows_v, sem):
        cid = lax.axis_index("c")
        sid = lax.axis_index("s")
        wid = sid * info.num_cores + cid
        for j in range(nch):
            base = wid * bpw + j * chunk
            pltpu.sync_copy(idx_hbm.at[pl.ds(base, chunk)], idx_v)
            pltpu.async_copy(table_hbm.at[idx_v], rows_v, sem).wait()
            pltpu.sync_copy(rows_v, out_hbm.at[pl.ds(base, chunk)])

    return gather


def _st_loss_body(x_ref, q_ref, out_ref, loss_ref):
    xb = x_ref[0]
    qt = q_ref[...].T
    out_ref[0] = xb + (qt - xb)
    d = xb - qt

    @pl.when((pl.program_id(0) == 0) & (pl.program_id(1) == 0))
    def _():
        loss_ref[...] = jnp.zeros((1, 1), jnp.float32)

    loss_ref[...] += jnp.sum(d * d).reshape(1, 1)


def _st_loss(xr, quant, bn):
    b, c, hw = xr.shape
    per_b = hw // bn
    return pl.pallas_call(
        _st_loss_body,
        grid=(b, per_b),
        in_specs=[
            pl.BlockSpec((1, c, bn), lambda i, j: (i, 0, j)),
            pl.BlockSpec((bn, c), lambda i, j: (i * per_b + j, 0)),
        ],
        out_specs=[
            pl.BlockSpec((1, c, bn), lambda i, j: (i, 0, j)),
            pl.BlockSpec((1, 1), lambda i, j: (0, 0)),
        ],
        out_shape=[
            jax.ShapeDtypeStruct((b, c, hw), jnp.float32),
            jax.ShapeDtypeStruct((1, 1), jnp.float32),
        ],
    )(xr, quant)


def kernel(x, codebook):
    b, c, h, w = x.shape
    k = codebook.shape[0]
    hw = h * w
    n = b * hw
    bn = 256

    xr = x.reshape(b, c, hw)
    xp = jnp.transpose(x, (0, 2, 3, 1)).reshape(b, hw, c)
    x2 = jnp.sum(xp * xp, axis=-1).reshape(n // bn, 1, bn)
    c2 = jnp.sum(codebook * codebook, axis=-1).reshape(k, 1)
    cb16 = codebook.astype(jnp.bfloat16)
    idx3 = _scores_argmin(xr, cb16, x2, c2, bn)
    idx_flat = idx3.reshape(n)
    quant = _make_codebook_gather(k, c, n)(codebook, idx_flat)
    quant_t, loss_sum = _st_loss(xr, quant, bn)
    loss = loss_sum[0, 0] / jnp.float32(n * c)
    quant_out = quant_t.reshape(b, c, h, w)
    return quant_out, loss, loss, idx_flat.reshape(b, hw)


# speedup vs baseline: 1.6537x; 1.6537x over previous
import functools

import jax
import jax.numpy as jnp
from jax import lax
from jax.experimental import pallas as pl
from jax.experimental.pallas import tpu as pltpu
from jax.experimental.pallas import tpu_sc as plsc


def _scores_argmin_body(x_ref, cb_ref, x2_ref, c2_ref, idx_ref, d2_ref):
    xb = x_ref[0]
    xb16 = xb.astype(jnp.bfloat16)
    cb = cb_ref[...]
    k = cb.shape[0]
    bn = xb.shape[1]
    x2 = x2_ref[0]
    kc = 32
    c2 = c2_ref[...]

    s2 = lax.dot_general(cb, xb16,
                         dimension_numbers=(((1,), (0,)), ((), ())),
                         preferred_element_type=jnp.float32)
    m2 = jnp.full((kc, bn), jnp.inf, jnp.float32)
    for i in range(k // kc):
        sc = s2[i * kc:(i + 1) * kc, :]
        cc = c2[i * kc:(i + 1) * kc, :]
        d2 = jnp.maximum((x2 - sc) + cc, 0.0)
        d2_ref[i * kc:(i + 1) * kc, :] = d2
        m2 = jnp.minimum(m2, d2)
    m2c = jnp.min(m2, axis=0, keepdims=True)

    dmin = jnp.sqrt(m2c)
    jio = lax.broadcasted_iota(jnp.int32, (16, bn), 0)
    cand_b = lax.bitcast_convert_type(m2c, jnp.int32) + jio
    cand = lax.bitcast_convert_type(cand_b, jnp.float32)
    ok = jnp.sqrt(cand) <= dmin
    thr = jnp.max(jnp.where(ok, cand, 0.0), axis=0, keepdims=True)

    bigf = jnp.float32(k)
    riof = lax.broadcasted_iota(jnp.int32, (kc, bn), 0).astype(jnp.float32)
    bestf = jnp.full((kc, bn), bigf, jnp.float32)
    for i in range(k // kc):
        cand = jnp.where(d2_ref[i * kc:(i + 1) * kc, :] <= thr,
                         riof + jnp.float32(i * kc), bigf)
        bestf = jnp.minimum(bestf, cand)
    idx_ref[0, 0, :] = jnp.min(bestf, axis=0).astype(jnp.int32)


def _scores_argmin(xr, cb16, x2, c2, bn):
    b, c, hw = xr.shape
    k = cb16.shape[0]
    nblk = (b * hw) // bn
    per_b = hw // bn
    return pl.pallas_call(
        _scores_argmin_body,
        grid=(nblk,),
        in_specs=[
            pl.BlockSpec((1, c, bn), lambda i: (i // per_b, 0, i % per_b)),
            pl.BlockSpec((k, c), lambda i: (0, 0)),
            pl.BlockSpec((1, 1, bn), lambda i: (i, 0, 0)),
            pl.BlockSpec((k, 1), lambda i: (0, 0)),
        ],
        out_specs=pl.BlockSpec((1, 1, bn), lambda i: (i, 0, 0)),
        out_shape=jax.ShapeDtypeStruct((nblk, 1, bn), jnp.int32),
        scratch_shapes=[pltpu.VMEM((k, bn), jnp.float32)],
    )(xr, cb16, x2, c2)


def _make_codebook_gather(k, c, n):
    info = plsc.get_sparse_core_info()
    nw = info.num_cores * info.num_subcores
    bpw = n // nw
    chunk = 128
    nch = bpw // chunk
    mesh = plsc.VectorSubcoreMesh(core_axis_name="c", subcore_axis_name="s")

    @functools.partial(
        pl.kernel,
        mesh=mesh,
        out_type=jax.ShapeDtypeStruct((n, c), jnp.float32),
        scratch_types=[
            pltpu.VMEM((chunk,), jnp.int32),
            pltpu.VMEM((chunk, c), jnp.float32),
            pltpu.SemaphoreType.DMA,
        ],
    )
    def gather(table_hbm, idx_hbm, out_hbm, idx_v, rows_v, sem):
        cid = lax.axis_index("c")
        sid = lax.axis_index("s")
        wid = sid * info.num_cores + cid
        for j in range(nch):
            base = wid * bpw + j * chunk
            pltpu.sync_copy(idx_hbm.at[pl.ds(base, chunk)], idx_v)
            pltpu.async_copy(table_hbm.at[idx_v], rows_v, sem).wait()
            pltpu.sync_copy(rows_v, out_hbm.at[pl.ds(base, chunk)])

    return gather


def _st_loss_body(x_ref, q_ref, out_ref, loss_ref):
    xb = x_ref[0]
    qt = q_ref[...].T
    out_ref[0] = xb + (qt - xb)
    d = xb - qt

    @pl.when((pl.program_id(0) == 0) & (pl.program_id(1) == 0))
    def _():
        loss_ref[...] = jnp.zeros((1, 1), jnp.float32)

    loss_ref[...] += jnp.sum(d * d).reshape(1, 1)


def _st_loss(xr, quant, bn):
    b, c, hw = xr.shape
    per_b = hw // bn
    return pl.pallas_call(
        _st_loss_body,
        grid=(b, per_b),
        in_specs=[
            pl.BlockSpec((1, c, bn), lambda i, j: (i, 0, j)),
            pl.BlockSpec((bn, c), lambda i, j: (i * per_b + j, 0)),
        ],
        out_specs=[
            pl.BlockSpec((1, c, bn), lambda i, j: (i, 0, j)),
            pl.BlockSpec((1, 1), lambda i, j: (0, 0)),
        ],
        out_shape=[
            jax.ShapeDtypeStruct((b, c, hw), jnp.float32),
            jax.ShapeDtypeStruct((1, 1), jnp.float32),
        ],
    )(xr, quant)


def kernel(x, codebook):
    b, c, h, w = x.shape
    k = codebook.shape[0]
    hw = h * w
    n = b * hw
    bn = 1024

    xr = x.reshape(b, c, hw)
    xp = jnp.transpose(x, (0, 2, 3, 1)).reshape(b, hw, c)
    x2 = jnp.sum(xp * xp, axis=-1).reshape(n // bn, 1, bn)
    c2 = jnp.sum(codebook * codebook, axis=-1).reshape(k, 1)
    cb16 = codebook.astype(jnp.bfloat16) * jnp.bfloat16(2.0)
    idx3 = _scores_argmin(xr, cb16, x2, c2, bn)
    idx_flat = idx3.reshape(n)
    quant = _make_codebook_gather(k, c, n)(codebook, idx_flat)
    quant_t, loss_sum = _st_loss(xr, quant, bn)
    loss = loss_sum[0, 0] / jnp.float32(n * c)
    quant_out = quant_t.reshape(b, c, h, w)
    return quant_out, loss, loss, idx_flat.reshape(b, hw)
